# Initial kernel scaffold; baseline (speedup 1.0000x reference)
import jax
import jax.numpy as jnp
from jax.experimental import pallas as pl


def kernel(t, beta, alpha):
    raise NotImplementedError("write your pallas kernel here")


# speedup vs baseline: 8.3798x; 8.3798x over previous
import functools

import jax
import jax.numpy as jnp
from jax import lax
from jax.experimental import pallas as pl
from jax.experimental.pallas import tpu as pltpu
from jax.experimental.pallas import tpu_sc as plsc

NUM_TIME_STEPS = 1000
BATCH = 16384
NUM_CORES = 2
NUM_SUBCORES = 16
LANES = 16
NUM_WORKERS = NUM_CORES * NUM_SUBCORES
B_PER_W = BATCH // NUM_WORKERS

_mesh = plsc.VectorSubcoreMesh(core_axis_name="c", subcore_axis_name="s")


@functools.partial(
    pl.kernel,
    mesh=_mesh,
    compiler_params=pltpu.CompilerParams(needs_layout_passes=False),
    out_type=(
        jax.ShapeDtypeStruct((BATCH,), jnp.float32),
        jax.ShapeDtypeStruct((BATCH,), jnp.float32),
    ),
    scratch_types=[
        pltpu.VMEM((B_PER_W,), jnp.int32),
        pltpu.VMEM((NUM_TIME_STEPS,), jnp.float32),
        pltpu.VMEM((NUM_TIME_STEPS,), jnp.float32),
        pltpu.VMEM((B_PER_W,), jnp.float32),
        pltpu.VMEM((B_PER_W,), jnp.float32),
    ],
)
def _ddpm_gather(t_hbm, beta_hbm, alpha_hbm, beta_out, alpha_out,
                 idx_v, beta_v, alpha_v, bout_v, aout_v):
    wid = lax.axis_index("s") * NUM_CORES + lax.axis_index("c")
    base = wid * B_PER_W

    pltpu.sync_copy(t_hbm.at[pl.ds(base, B_PER_W)], idx_v)
    pltpu.sync_copy(beta_hbm, beta_v)
    pltpu.sync_copy(alpha_hbm, alpha_v)

    for i in range(B_PER_W // LANES):
        idx = idx_v[pl.ds(i * LANES, LANES)]
        bout_v[pl.ds(i * LANES, LANES)] = plsc.load_gather(beta_v, [idx])
        aout_v[pl.ds(i * LANES, LANES)] = plsc.load_gather(alpha_v, [idx])

    pltpu.sync_copy(bout_v, beta_out.at[pl.ds(base, B_PER_W)])
    pltpu.sync_copy(aout_v, alpha_out.at[pl.ds(base, B_PER_W)])


def kernel(t, beta, alpha):
    return _ddpm_gather(t, beta, alpha)


# speedup vs baseline: 8.7750x; 1.0472x over previous
import functools

import jax
import jax.numpy as jnp
from jax import lax
from jax.experimental import pallas as pl
from jax.experimental.pallas import tpu as pltpu
from jax.experimental.pallas import tpu_sc as plsc

NUM_TIME_STEPS = 1000
BATCH = 16384
NUM_CORES = 2
NUM_SUBCORES = 16
LANES = 16
NUM_WORKERS = NUM_CORES * NUM_SUBCORES
B_PER_W = BATCH // NUM_WORKERS

_mesh = plsc.VectorSubcoreMesh(core_axis_name="c", subcore_axis_name="s")


@functools.partial(
    pl.kernel,
    mesh=_mesh,
    compiler_params=pltpu.CompilerParams(needs_layout_passes=False),
    out_type=(
        jax.ShapeDtypeStruct((BATCH,), jnp.float32),
        jax.ShapeDtypeStruct((BATCH,), jnp.float32),
    ),
    scratch_types=[
        pltpu.VMEM((B_PER_W,), jnp.int32),
        pltpu.VMEM((NUM_TIME_STEPS,), jnp.float32),
        pltpu.VMEM((NUM_TIME_STEPS,), jnp.float32),
        pltpu.VMEM((B_PER_W,), jnp.float32),
        pltpu.VMEM((B_PER_W,), jnp.float32),
        pltpu.SemaphoreType.DMA,
    ],
)
def _ddpm_gather(t_hbm, beta_hbm, alpha_hbm, beta_out, alpha_out,
                 idx_v, beta_v, alpha_v, bout_v, aout_v, sem):
    wid = lax.axis_index("s") * NUM_CORES + lax.axis_index("c")
    base = wid * B_PER_W

    in0 = pltpu.async_copy(t_hbm.at[pl.ds(base, B_PER_W)], idx_v, sem)
    in1 = pltpu.async_copy(beta_hbm, beta_v, sem)
    in2 = pltpu.async_copy(alpha_hbm, alpha_v, sem)
    in0.wait()
    in1.wait()
    in2.wait()

    for i in range(B_PER_W // LANES):
        idx = idx_v[pl.ds(i * LANES, LANES)]
        bout_v[pl.ds(i * LANES, LANES)] = plsc.load_gather(beta_v, [idx])
        aout_v[pl.ds(i * LANES, LANES)] = plsc.load_gather(alpha_v, [idx])

    out0 = pltpu.async_copy(bout_v, beta_out.at[pl.ds(base, B_PER_W)], sem)
    out1 = pltpu.async_copy(aout_v, alpha_out.at[pl.ds(base, B_PER_W)], sem)
    out0.wait()
    out1.wait()


def kernel(t, beta, alpha):
    return _ddpm_gather(t, beta, alpha)
